# baseline (speedup 1.0000x reference)
import functools

import jax
import jax.numpy as jnp
from jax import lax
from jax.experimental import pallas as pl
from jax.experimental.pallas import tpu as pltpu
from jax.experimental.pallas import tpu_sc as plsc

B, NQ, N, DIN, DIM, K = 4, 2048, 4096, 128, 64, 16
NFREQ = 5
TROW = 160
TQ_KNN = 256
TN_TAB = 1024
TQ_ATT = 256
R_ATT = TQ_ATT * K
CH_SC = 128
NW = 32


def _table_kernel(pts_ref, xyz_ref, wkvq_ref, waux_ref, f10_ref, f5_ref,
                  out_ref):
    f32 = jnp.float32
    hp = jnp.float32(jnp.pi / 2)

    def mm(a, w):
        return jax.lax.dot_general(a, w, (((1,), (0,)), ((), ())),
                                   preferred_element_type=f32)

    kv = mm(pts_ref[...], wkvq_ref[:, 0:2 * DIM])
    xyz = xyz_ref[...]
    xgx = xyz[:, 0:1]
    xgy = xyz[:, 1:2]
    xgz = xyz[:, 2:3]
    axy = jnp.concatenate([jnp.broadcast_to(xgx, (TN_TAB, NFREQ)),
                           jnp.broadcast_to(xgy, (TN_TAB, NFREQ))],
                          axis=1) * f10_ref[...]
    az = jnp.broadcast_to(xgz, (TN_TAB, NFREQ)) * f5_ref[...]
    t30 = jnp.sin(jnp.concatenate([axy, axy + hp, az, az + hp], axis=1))
    t20 = t30[:, 0:20]
    ptfeat = jnp.concatenate([xgz, t30[:, 20:30], xgx, xgy], axis=1)
    pep = mm(ptfeat, waux_ref[0:13, :])
    out_ref[:, 0:DIM] = kv[:, 0:DIM] - pep
    out_ref[:, DIM:2 * DIM] = kv[:, DIM:2 * DIM] + pep
    out_ref[:, 2 * DIM:2 * DIM + 20] = t20
    out_ref[:, 2 * DIM + 20:] = jnp.zeros((TN_TAB, TROW - 2 * DIM - 20), f32)


def _build_table(points2d, xyz2d, wkvq, waux, f10, f5):
    grid = (points2d.shape[0] // TN_TAB,)
    const = lambda p: (0, 0)
    return pl.pallas_call(
        _table_kernel,
        grid=grid,
        in_specs=[
            pl.BlockSpec((TN_TAB, DIN), lambda p: (p, 0)),
            pl.BlockSpec((TN_TAB, 3), lambda p: (p, 0)),
            pl.BlockSpec((DIN, 5 * DIM), const),
            pl.BlockSpec((35, DIM), const),
            pl.BlockSpec((1, 10), const),
            pl.BlockSpec((1, NFREQ), const),
        ],
        out_specs=pl.BlockSpec((TN_TAB, TROW), lambda p: (p, 0)),
        out_shape=jax.ShapeDtypeStruct((points2d.shape[0], TROW), jnp.float32),
    )(points2d, xyz2d, wkvq, waux, f10, f5)


def _knn_kernel(xq_ref, xt_ref, out_ref):
    p = pl.program_id(0)
    b = p // (NQ // TQ_KNN)
    xq = xq_ref[...]
    xrow = xt_ref[0, 0:1, :]
    yrow = xt_ref[0, 1:2, :]
    qx = xq[:, 0:1]
    qy = xq[:, 1:2]
    d = (qx - xrow) ** 2 + (qy - yrow) ** 2
    iota = lax.broadcasted_iota(jnp.int32, (TQ_KNN, N), 1).astype(jnp.float32)
    base = jnp.int32(N) * b
    for j in range(K):
        m = jnp.min(d, axis=1, keepdims=True)
        cand = jnp.where(d == m, iota, jnp.float32(N))
        idx = jnp.min(cand, axis=1, keepdims=True)
        out_ref[:, j:j + 1] = idx.astype(jnp.int32) + base
        d = jnp.where(iota == idx, jnp.inf, d)


def _knn(xq2d, xyz_t):
    grid = (xq2d.shape[0] // TQ_KNN,)
    return pl.pallas_call(
        _knn_kernel,
        grid=grid,
        in_specs=[
            pl.BlockSpec((TQ_KNN, 2), lambda p: (p, 0)),
            pl.BlockSpec((1, 2, N), lambda p: (p // (NQ // TQ_KNN), 0, 0)),
        ],
        out_specs=pl.BlockSpec((TQ_KNN, K), lambda p: (p, 0)),
        out_shape=jax.ShapeDtypeStruct((xq2d.shape[0], K), jnp.int32),
    )(xq2d, xyz_t)


def _sc_gather(table, idx_flat):
    nrows = idx_flat.shape[0]
    rows_per_w = nrows // NW
    n_chunks = rows_per_w // CH_SC
    mesh = plsc.VectorSubcoreMesh(core_axis_name="c", subcore_axis_name="s")

    @functools.partial(
        pl.kernel, mesh=mesh,
        out_type=jax.ShapeDtypeStruct((nrows, TROW), jnp.float32),
        scratch_types=[
            pltpu.VMEM((rows_per_w,), jnp.int32),
            pltpu.VMEM((CH_SC, TROW), jnp.float32),
            pltpu.VMEM((CH_SC, TROW), jnp.float32),
            pltpu.SemaphoreType.DMA,
            pltpu.SemaphoreType.DMA,
            pltpu.SemaphoreType.DMA,
            pltpu.SemaphoreType.DMA,
        ],
        compiler_params=pltpu.CompilerParams(use_tc_tiling_on_sc=False),
    )
    def gather_k(table_hbm, idx_hbm, out_hbm, idx_v, rows0, rows1,
                 g0, g1, o0, o1):
        wid = lax.axis_index("s") * 2 + lax.axis_index("c")
        base = wid * rows_per_w
        pltpu.sync_copy(idx_hbm.at[pl.ds(base, rows_per_w)], idx_v)
        rows = (rows0, rows1)
        gsem = (g0, g1)
        osem = (o0, o1)
        gh = {}
        oh = {}
        gh[0] = pltpu.async_copy(
            table_hbm.at[idx_v.at[pl.ds(0, CH_SC)]], rows0, g0)
        for c in range(n_chunks):
            p = c & 1
            nxt = (c + 1) & 1
            if c + 1 < n_chunks:
                if c >= 1:
                    oh[c - 1].wait()
                gh[c + 1] = pltpu.async_copy(
                    table_hbm.at[idx_v.at[pl.ds((c + 1) * CH_SC, CH_SC)]],
                    rows[nxt], gsem[nxt])
            gh[c].wait()
            oh[c] = pltpu.async_copy(
                rows[p], out_hbm.at[pl.ds(base + c * CH_SC, CH_SC)], osem[p])
        oh[n_chunks - 2].wait()
        oh[n_chunks - 1].wait()

    return gather_k(table, idx_flat)


def _attn_kernel(g_ref, lat_ref, xq_ref, wkvq_ref, wg1_ref, wg2_ref,
                 waux_ref, bias_ref, f10_ref, out_ref):
    f32 = jnp.float32
    hp = jnp.float32(jnp.pi / 2)

    def mm(a, w):
        return jax.lax.dot_general(a, w, (((1,), (0,)), ((), ())),
                                   preferred_element_type=f32)

    def rep(x):
        w = x.shape[-1]
        return jnp.broadcast_to(x[:, None, :], (TQ_ATT, K, w)).reshape(
            R_ATT, w)

    g = g_ref[...]
    kn = g[:, 0:DIM]
    vn = g[:, DIM:2 * DIM]
    gs2 = jnp.concatenate([g[:, 2 * DIM:2 * DIM + 10],
                           g[:, 2 * DIM:2 * DIM + 10]], axis=1)
    gc2 = jnp.concatenate([g[:, 2 * DIM + 10:2 * DIM + 20],
                           g[:, 2 * DIM + 10:2 * DIM + 20]], axis=1)

    lat = lat_ref[...]
    qkv = mm(lat, wkvq_ref[:, 2 * DIM:5 * DIM])
    q = qkv[:, 0:DIM]
    kg = qkv[:, DIM:2 * DIM]
    vg = qkv[:, 2 * DIM:3 * DIM]

    xq = xq_ref[...]
    aq = jnp.concatenate([jnp.broadcast_to(xq[:, 0:1], (TQ_ATT, NFREQ)),
                          jnp.broadcast_to(xq[:, 1:2], (TQ_ATT, NFREQ))],
                         axis=1) * f10_ref[...]
    qt20 = jnp.sin(jnp.concatenate([aq, aq + hp], axis=1))
    qalt = jnp.concatenate([-qt20[:, 10:20], qt20[:, 0:10]], axis=1)
    peq = mm(xq, waux_ref[33:35, :]) + bias_ref[2:3, :]
    q2 = q + peq

    feat20 = rep(qt20) * gc2 + rep(qalt) * gs2
    pex = mm(feat20, waux_ref[13:33, :])

    bg1 = bias_ref[0:1, :]
    bg2 = bias_ref[1:2, :]
    wg1 = wg1_ref[...]
    wg2 = wg2_ref[...]
    h = rep(q2) - kn + pex
    gam_n = mm(jnp.maximum(mm(h, wg1) + bg1, 0.0), wg2) + bg2
    h_g = q - kg
    gam_g = mm(jnp.maximum(mm(h_g, wg1) + bg1, 0.0), wg2) + bg2

    gn3 = gam_n.reshape(TQ_ATT, K, DIM)
    m = jnp.maximum(jnp.max(gn3, axis=1), gam_g)
    sn = jnp.exp(gn3 - m[:, None, :])
    sg = jnp.exp(gam_g - m)
    w3 = (vn + pex + rep(peq)).reshape(TQ_ATT, K, DIM)
    wsum = jnp.sum(sn * w3, axis=1)
    denom = jnp.sum(sn, axis=1) + sg
    out_ref[...] = (wsum + sg * vg) / denom


def _attention(gathered, lat2d, xq2d, wkvq, wg1, wg2, waux, bias3, f10):
    grid = (lat2d.shape[0] // TQ_ATT,)
    const = lambda p: (0, 0)
    return pl.pallas_call(
        _attn_kernel,
        grid=grid,
        in_specs=[
            pl.BlockSpec((R_ATT, TROW), lambda p: (p, 0)),
            pl.BlockSpec((TQ_ATT, DIN), lambda p: (p, 0)),
            pl.BlockSpec((TQ_ATT, 2), lambda p: (p, 0)),
            pl.BlockSpec((DIN, 5 * DIM), const),
            pl.BlockSpec((DIM, DIM), const),
            pl.BlockSpec((DIM, DIM), const),
            pl.BlockSpec((35, DIM), const),
            pl.BlockSpec((3, DIM), const),
            pl.BlockSpec((1, 10), const),
        ],
        out_specs=pl.BlockSpec((TQ_ATT, DIM), lambda p: (p, 0)),
        out_shape=jax.ShapeDtypeStruct((lat2d.shape[0], DIM), jnp.float32),
    )(gathered, lat2d, xq2d, wkvq, wg1, wg2, waux, bias3, f10)


def kernel(xyz_q, lat_rep, xyz, points, Wq, Wkg, Wvg, Wk, Wv, Wg1, bg1,
           Wg2, bg2, Wpe, bpe):
    f32 = jnp.float32
    points2d = points.reshape(B * N, DIN)
    xyz2d = xyz.reshape(B * N, 3)
    xq2d = xyz_q.reshape(B * NQ, 2)
    lat2d = lat_rep.reshape(B * NQ, DIN)
    xyz_t = jnp.transpose(xyz[:, :, :2], (0, 2, 1))

    wkvq = jnp.concatenate([Wk, Wv, Wq, Wkg, Wvg], axis=1)

    fb = jnp.linspace(1.0, 2.0 ** NFREQ, NFREQ).astype(f32)
    f5 = fb.reshape(1, NFREQ)
    f10 = jnp.concatenate([fb, fb]).reshape(1, 2 * NFREQ)
    sgn = jnp.ones((33, 1), f32).at[0:2].set(-1.0)
    perm = jnp.array(
        [2] + [3 + 6 * i + 2 for i in range(NFREQ)]
        + [6 + 6 * i + 2 for i in range(NFREQ)] + [0, 1]
        + [3 + 6 * i + c for c in (0, 1) for i in range(NFREQ)]
        + [6 + 6 * i + c for c in (0, 1) for i in range(NFREQ)] + [0, 1])
    scale = jnp.ones((35, 1), f32).at[11:13].set(-1.0)
    waux = Wpe[perm] * scale
    bias3 = jnp.stack([bg1, bg2, bpe])

    H = B // 2
    outs = []
    for h in range(2):
        pr = points2d[h * H * N:(h + 1) * H * N]
        xr = xyz2d[h * H * N:(h + 1) * H * N]
        xqh = xq2d[h * H * NQ:(h + 1) * H * NQ]
        lath = lat2d[h * H * NQ:(h + 1) * H * NQ]
        xth = xyz_t[h * H:(h + 1) * H]
        th = _build_table(pr, xr, wkvq, waux, f10, f5)
        kh = _knn(xqh, xth)
        gh = _sc_gather(th, kh.reshape(H * NQ * K))
        outs.append(_attention(gh, lath, xqh, wkvq, Wg1, Wg2, waux, bias3,
                               f10))
    return jnp.concatenate(outs).reshape(B, NQ, DIM)


# speedup vs baseline: 1.0177x; 1.0177x over previous
import functools

import jax
import jax.numpy as jnp
from jax import lax
from jax.experimental import pallas as pl
from jax.experimental.pallas import tpu as pltpu
from jax.experimental.pallas import tpu_sc as plsc

B, NQ, N, DIN, DIM, K = 4, 2048, 4096, 128, 64, 16
NFREQ = 5
TROW = 160
TQ_KNN = 256
TN_TAB = 1024
TQ_ATT = 256
R_ATT = TQ_ATT * K
CH_SC = 128
NW = 32


def _table_kernel(pts_ref, xyz_ref, wkvq_ref, waux_ref, f10_ref, f5_ref,
                  out_ref):
    f32 = jnp.float32
    hp = jnp.float32(jnp.pi / 2)

    def mm(a, w):
        return jax.lax.dot_general(a, w, (((1,), (0,)), ((), ())),
                                   preferred_element_type=f32)

    kv = mm(pts_ref[...], wkvq_ref[:, 0:2 * DIM])
    xyz = xyz_ref[...]
    xgx = xyz[:, 0:1]
    xgy = xyz[:, 1:2]
    xgz = xyz[:, 2:3]
    axy = jnp.concatenate([jnp.broadcast_to(xgx, (TN_TAB, NFREQ)),
                           jnp.broadcast_to(xgy, (TN_TAB, NFREQ))],
                          axis=1) * f10_ref[...]
    az = jnp.broadcast_to(xgz, (TN_TAB, NFREQ)) * f5_ref[...]
    t30 = jnp.sin(jnp.concatenate([axy, axy + hp, az, az + hp], axis=1))
    t20 = t30[:, 0:20]
    ptfeat = jnp.concatenate([xgz, t30[:, 20:30], xgx, xgy], axis=1)
    pep = mm(ptfeat, waux_ref[0:13, :])
    out_ref[:, 0:DIM] = kv[:, 0:DIM] - pep
    out_ref[:, DIM:2 * DIM] = kv[:, DIM:2 * DIM] + pep
    out_ref[:, 2 * DIM:2 * DIM + 20] = t20
    out_ref[:, 2 * DIM + 20:] = jnp.zeros((TN_TAB, TROW - 2 * DIM - 20), f32)


def _build_table(points2d, xyz2d, wkvq, waux, f10, f5):
    grid = (B * N // TN_TAB,)
    const = lambda p: (0, 0)
    return pl.pallas_call(
        _table_kernel,
        grid=grid,
        in_specs=[
            pl.BlockSpec((TN_TAB, DIN), lambda p: (p, 0)),
            pl.BlockSpec((TN_TAB, 3), lambda p: (p, 0)),
            pl.BlockSpec((DIN, 5 * DIM), const),
            pl.BlockSpec((35, DIM), const),
            pl.BlockSpec((1, 10), const),
            pl.BlockSpec((1, NFREQ), const),
        ],
        out_specs=pl.BlockSpec((TN_TAB, TROW), lambda p: (p, 0)),
        out_shape=jax.ShapeDtypeStruct((B * N, TROW), jnp.float32),
    )(points2d, xyz2d, wkvq, waux, f10, f5)


def _knn_kernel(xq_ref, xt_ref, out_ref):
    p = pl.program_id(0)
    b = p // (NQ // TQ_KNN)
    xq = xq_ref[...]
    xrow = xt_ref[0, 0:1, :]
    yrow = xt_ref[0, 1:2, :]
    qx = xq[:, 0:1]
    qy = xq[:, 1:2]
    d = (qx - xrow) ** 2 + (qy - yrow) ** 2
    iota = lax.broadcasted_iota(jnp.int32, (TQ_KNN, N), 1).astype(jnp.float32)
    base = jnp.int32(N) * b
    for j in range(K):
        m = jnp.min(d, axis=1, keepdims=True)
        cand = jnp.where(d == m, iota, jnp.float32(N))
        idx = jnp.min(cand, axis=1, keepdims=True)
        out_ref[:, j:j + 1] = idx.astype(jnp.int32) + base
        d = jnp.where(iota == idx, jnp.inf, d)


def _knn(xq2d, xyz_t):
    grid = (B * NQ // TQ_KNN,)
    return pl.pallas_call(
        _knn_kernel,
        grid=grid,
        in_specs=[
            pl.BlockSpec((TQ_KNN, 2), lambda p: (p, 0)),
            pl.BlockSpec((1, 2, N), lambda p: (p // (NQ // TQ_KNN), 0, 0)),
        ],
        out_specs=pl.BlockSpec((TQ_KNN, K), lambda p: (p, 0)),
        out_shape=jax.ShapeDtypeStruct((B * NQ, K), jnp.int32),
    )(xq2d, xyz_t)


def _sc_gather(table, idx_flat):
    rows_per_w = (B * NQ * K) // NW
    n_chunks = rows_per_w // CH_SC
    mesh = plsc.VectorSubcoreMesh(core_axis_name="c", subcore_axis_name="s")

    @functools.partial(
        pl.kernel, mesh=mesh,
        out_type=jax.ShapeDtypeStruct((B * NQ * K, TROW), jnp.float32),
        scratch_types=[
            pltpu.VMEM((rows_per_w,), jnp.int32),
            pltpu.VMEM((CH_SC, TROW), jnp.float32),
            pltpu.VMEM((CH_SC, TROW), jnp.float32),
            pltpu.SemaphoreType.DMA,
            pltpu.SemaphoreType.DMA,
            pltpu.SemaphoreType.DMA,
            pltpu.SemaphoreType.DMA,
        ],
        compiler_params=pltpu.CompilerParams(use_tc_tiling_on_sc=False),
    )
    def gather_k(table_hbm, idx_hbm, out_hbm, idx_v, rows0, rows1,
                 g0, g1, o0, o1):
        wid = lax.axis_index("s") * 2 + lax.axis_index("c")
        base = wid * rows_per_w
        pltpu.sync_copy(idx_hbm.at[pl.ds(base, rows_per_w)], idx_v)
        rows = (rows0, rows1)
        gsem = (g0, g1)
        osem = (o0, o1)
        gh = {}
        oh = {}
        gh[0] = pltpu.async_copy(
            table_hbm.at[idx_v.at[pl.ds(0, CH_SC)]], rows0, g0)
        for c in range(n_chunks):
            p = c & 1
            nxt = (c + 1) & 1
            if c + 1 < n_chunks:
                if c >= 1:
                    oh[c - 1].wait()
                gh[c + 1] = pltpu.async_copy(
                    table_hbm.at[idx_v.at[pl.ds((c + 1) * CH_SC, CH_SC)]],
                    rows[nxt], gsem[nxt])
            gh[c].wait()
            oh[c] = pltpu.async_copy(
                rows[p], out_hbm.at[pl.ds(base + c * CH_SC, CH_SC)], osem[p])
        oh[n_chunks - 2].wait()
        oh[n_chunks - 1].wait()

    return gather_k(table, idx_flat)


def _attn_kernel(g_ref, lat_ref, xq_ref, wkvq_ref, wg1_ref, wg2_ref,
                 waux_ref, bias_ref, f10_ref, out_ref):
    f32 = jnp.float32
    hp = jnp.float32(jnp.pi / 2)

    def mm(a, w):
        return jax.lax.dot_general(a, w, (((1,), (0,)), ((), ())),
                                   preferred_element_type=f32)

    def rep(x):
        w = x.shape[-1]
        return jnp.broadcast_to(x[:, None, :], (TQ_ATT, K, w)).reshape(
            R_ATT, w)

    g = g_ref[...]
    kn = g[:, 0:DIM]
    vn = g[:, DIM:2 * DIM]
    gs2 = jnp.concatenate([g[:, 2 * DIM:2 * DIM + 10],
                           g[:, 2 * DIM:2 * DIM + 10]], axis=1)
    gc2 = jnp.concatenate([g[:, 2 * DIM + 10:2 * DIM + 20],
                           g[:, 2 * DIM + 10:2 * DIM + 20]], axis=1)

    lat = lat_ref[...]
    qkv = mm(lat, wkvq_ref[:, 2 * DIM:5 * DIM])
    q = qkv[:, 0:DIM]
    kg = qkv[:, DIM:2 * DIM]
    vg = qkv[:, 2 * DIM:3 * DIM]

    xq = xq_ref[...]
    aq = jnp.concatenate([jnp.broadcast_to(xq[:, 0:1], (TQ_ATT, NFREQ)),
                          jnp.broadcast_to(xq[:, 1:2], (TQ_ATT, NFREQ))],
                         axis=1) * f10_ref[...]
    qt20 = jnp.sin(jnp.concatenate([aq, aq + hp], axis=1))
    qalt = jnp.concatenate([-qt20[:, 10:20], qt20[:, 0:10]], axis=1)
    peq = mm(xq, waux_ref[33:35, :]) + bias_ref[2:3, :]
    q2 = q + peq

    feat20 = rep(qt20) * gc2 + rep(qalt) * gs2
    pex = mm(feat20, waux_ref[13:33, :])

    bg1 = bias_ref[0:1, :]
    bg2 = bias_ref[1:2, :]
    wg1 = wg1_ref[...]
    wg2 = wg2_ref[...]
    h = rep(q2) - kn + pex
    gam_n = mm(jnp.maximum(mm(h, wg1) + bg1, 0.0), wg2) + bg2
    h_g = q - kg
    gam_g = mm(jnp.maximum(mm(h_g, wg1) + bg1, 0.0), wg2) + bg2

    gn3 = gam_n.reshape(TQ_ATT, K, DIM)
    m = jnp.maximum(jnp.max(gn3, axis=1), gam_g)
    sn = jnp.exp(gn3 - m[:, None, :])
    sg = jnp.exp(gam_g - m)
    w3 = (vn + pex + rep(peq)).reshape(TQ_ATT, K, DIM)
    wsum = jnp.sum(sn * w3, axis=1)
    denom = jnp.sum(sn, axis=1) + sg
    out_ref[...] = (wsum + sg * vg) / denom


def _attention(gathered, lat2d, xq2d, wkvq, wg1, wg2, waux, bias3, f10):
    grid = (B * NQ // TQ_ATT,)
    const = lambda p: (0, 0)
    return pl.pallas_call(
        _attn_kernel,
        grid=grid,
        in_specs=[
            pl.BlockSpec((R_ATT, TROW), lambda p: (p, 0)),
            pl.BlockSpec((TQ_ATT, DIN), lambda p: (p, 0)),
            pl.BlockSpec((TQ_ATT, 2), lambda p: (p, 0)),
            pl.BlockSpec((DIN, 5 * DIM), const),
            pl.BlockSpec((DIM, DIM), const),
            pl.BlockSpec((DIM, DIM), const),
            pl.BlockSpec((35, DIM), const),
            pl.BlockSpec((3, DIM), const),
            pl.BlockSpec((1, 10), const),
        ],
        out_specs=pl.BlockSpec((TQ_ATT, DIM), lambda p: (p, 0)),
        out_shape=jax.ShapeDtypeStruct((B * NQ, DIM), jnp.float32),
    )(gathered, lat2d, xq2d, wkvq, wg1, wg2, waux, bias3, f10)


def kernel(xyz_q, lat_rep, xyz, points, Wq, Wkg, Wvg, Wk, Wv, Wg1, bg1,
           Wg2, bg2, Wpe, bpe):
    f32 = jnp.float32
    points2d = points.reshape(B * N, DIN)
    xyz2d = xyz.reshape(B * N, 3)
    xq2d = xyz_q.reshape(B * NQ, 2)
    lat2d = lat_rep.reshape(B * NQ, DIN)
    xyz_t = jnp.transpose(xyz[:, :, :2], (0, 2, 1))

    wkvq = jnp.concatenate([Wk, Wv, Wq, Wkg, Wvg], axis=1)

    fb = jnp.linspace(1.0, 2.0 ** NFREQ, NFREQ).astype(f32)
    f5 = fb.reshape(1, NFREQ)
    f10 = jnp.concatenate([fb, fb]).reshape(1, 2 * NFREQ)
    sgn = jnp.ones((33, 1), f32).at[0:2].set(-1.0)
    perm = jnp.array(
        [2] + [3 + 6 * i + 2 for i in range(NFREQ)]
        + [6 + 6 * i + 2 for i in range(NFREQ)] + [0, 1]
        + [3 + 6 * i + c for c in (0, 1) for i in range(NFREQ)]
        + [6 + 6 * i + c for c in (0, 1) for i in range(NFREQ)] + [0, 1])
    scale = jnp.ones((35, 1), f32).at[11:13].set(-1.0)
    waux = Wpe[perm] * scale
    bias3 = jnp.stack([bg1, bg2, bpe])

    table = _build_table(points2d, xyz2d, wkvq, waux, f10, f5)
    knn = _knn(xq2d, xyz_t)
    gathered = _sc_gather(table, knn.reshape(B * NQ * K))
    res = _attention(gathered, lat2d, xq2d, wkvq, Wg1, Wg2, waux, bias3, f10)
    return res.reshape(B, NQ, DIM)
